# baseline (device time: 15682 ns/iter reference)
import jax
import jax.numpy as jnp
from jax import lax
from jax.experimental import pallas as pl
from jax.experimental.pallas import tpu as pltpu

B = 8
NBT = 64
BS = 16
H = 8
D = 64
NP_Q = 32
NLQ = NP_Q * BS * H
NEG = -1e30


def kernel(Q, K, V, bt, lens):
    def body(q_ref, k_ref, v_ref, bt_ref, lens_ref, out_ref,
             o_send, o_recv1, o_recv2, st_send, st_recv1, st_recv2,
             send_sems, recv_sems):
        my_x = lax.axis_index("x")
        my_y = lax.axis_index("y")
        nbr_y = (my_x, 1 - my_y)
        nbr_x = (1 - my_x, my_y)

        barrier_sem = pltpu.get_barrier_semaphore()
        for nbr in (nbr_y, nbr_x):
            pl.semaphore_signal(barrier_sem, inc=1, device_id=nbr,
                                device_id_type=pl.DeviceIdType.MESH)
        pl.semaphore_wait(barrier_sem, 2)

        bt_v = bt_ref[...]
        lens_v = lens_ref[...]
        j_iota = lax.broadcasted_iota(jnp.int32, (B, NBT), 1)
        valid_f = (j_iota < lens_v).astype(jnp.float32)
        off = my_x * 64 + my_y * NP_Q
        p_iota = lax.broadcasted_iota(jnp.int32, (B, NBT, NP_Q), 2) + off
        match_f = (bt_v[:, :, None] == p_iota).astype(jnp.float32)
        counts = jnp.sum(match_f * valid_f[:, :, None], axis=1)
        count3 = jnp.broadcast_to(
            counts[:, :, None], (B, NP_Q, BS * H)
        ).reshape(B, NLQ)
        eye_h = (
            lax.broadcasted_iota(jnp.int32, (H, NLQ), 0)
            == lax.broadcasted_iota(jnp.int32, (H, NLQ), 1) % H
        ).astype(jnp.float32)
        mask = count3[:, None, :] * eye_h[None, :, :]

        q2 = q_ref[...].reshape(B * H, D) * (D ** -0.5)
        k2 = k_ref[pl.ds(my_y * NP_Q, NP_Q)].reshape(NLQ, D)
        v3 = v_ref[pl.ds(my_y * NP_Q, NP_Q)].reshape(NLQ, D)

        s = lax.dot_general(
            q2, k2, (((1,), (1,)), ((), ())),
            preferred_element_type=jnp.float32,
        ).reshape(B, H, NLQ)
        s_masked = jnp.where(mask > 0, s, NEG)
        m = jnp.maximum(jnp.max(s_masked, axis=-1), NEG)
        p3 = mask * jnp.exp(s_masked - m[:, :, None])
        l = jnp.sum(p3, axis=-1)
        o = lax.dot_general(
            p3.reshape(B * H, NLQ), v3, (((1,), (0,)), ((), ())),
            preferred_element_type=jnp.float32,
        ).reshape(B, H, D)

        def combine(m0, l0, o0, m1, l1, o1):
            m_new = jnp.maximum(m0, m1)
            a0 = jnp.exp(m0 - m_new)
            a1 = jnp.exp(m1 - m_new)
            return (m_new, l0 * a0 + l1 * a1,
                    o0 * a0[:, :, None] + o1 * a1[:, :, None])

        def exchange(rnd, nbr, o_recv, st_recv, m, l, o):
            o_send[...] = o
            st_send[0, :, :] = m
            st_send[1, :, :] = l
            rdma_o = pltpu.make_async_remote_copy(
                src_ref=o_send, dst_ref=o_recv,
                send_sem=send_sems.at[2 * rnd], recv_sem=recv_sems.at[2 * rnd],
                device_id=nbr, device_id_type=pl.DeviceIdType.MESH,
            )
            rdma_st = pltpu.make_async_remote_copy(
                src_ref=st_send, dst_ref=st_recv,
                send_sem=send_sems.at[2 * rnd + 1],
                recv_sem=recv_sems.at[2 * rnd + 1],
                device_id=nbr, device_id_type=pl.DeviceIdType.MESH,
            )
            rdma_o.start()
            rdma_st.start()
            rdma_o.wait()
            rdma_st.wait()
            return combine(m, l, o,
                           st_recv[0, :, :], st_recv[1, :, :], o_recv[...])

        m, l, o = exchange(0, nbr_y, o_recv1, st_recv1, m, l, o)
        m, l, o = exchange(1, nbr_x, o_recv2, st_recv2, m, l, o)

        out_ref[...] = (o / l[:, :, None]).reshape(B, 1, H, D)

    return pl.pallas_call(
        body,
        out_shape=jax.ShapeDtypeStruct((B, 1, H, D), jnp.float32),
        in_specs=[pl.BlockSpec(memory_space=pltpu.VMEM)] * 5,
        out_specs=pl.BlockSpec(memory_space=pltpu.VMEM),
        scratch_shapes=[
            pltpu.VMEM((B, H, D), jnp.float32),
            pltpu.VMEM((B, H, D), jnp.float32),
            pltpu.VMEM((B, H, D), jnp.float32),
            pltpu.VMEM((2, B, H), jnp.float32),
            pltpu.VMEM((2, B, H), jnp.float32),
            pltpu.VMEM((2, B, H), jnp.float32),
            pltpu.SemaphoreType.DMA((4,)),
            pltpu.SemaphoreType.DMA((4,)),
        ],
        compiler_params=pltpu.CompilerParams(
            collective_id=0, vmem_limit_bytes=64 * 1024 * 1024
        ),
    )(Q, K, V, bt, lens.reshape(B, 1))


# device time: 13186 ns/iter; 1.1893x vs baseline; 1.1893x over previous
import jax
import jax.numpy as jnp
from jax import lax
from jax.experimental import pallas as pl
from jax.experimental.pallas import tpu as pltpu

B = 8
NBT = 64
BS = 16
H = 8
D = 64
NP_LOCAL = 64
NK = NP_LOCAL * BS
NL = NK * H
NEG = -1e30


def kernel(Q, K, V, bt, lens):
    def body(q_ref, k_ref, v_ref, bt_ref, lens_ref, out_ref,
             o_send, o_recv, st_send, st_recv, send_sems, recv_sems):
        my_x = lax.axis_index("x")
        my_y = lax.axis_index("y")
        nbr = (1 - my_x, my_y)

        barrier_sem = pltpu.get_barrier_semaphore()
        pl.semaphore_signal(barrier_sem, inc=1, device_id=nbr,
                            device_id_type=pl.DeviceIdType.MESH)
        pl.semaphore_wait(barrier_sem, 1)

        bt_v = bt_ref[...]
        lens_v = lens_ref[...]
        j_iota = lax.broadcasted_iota(jnp.int32, (B, NBT), 1)
        valid_f = (j_iota < lens_v).astype(jnp.float32)
        off = my_x * NP_LOCAL
        p_iota = lax.broadcasted_iota(jnp.int32, (B, NBT, NP_LOCAL), 2) + off
        match_f = (bt_v[:, :, None] == p_iota).astype(jnp.float32)
        counts = jnp.sum(match_f * valid_f[:, :, None], axis=1)
        count3 = jnp.broadcast_to(
            counts[:, :, None], (B, NP_LOCAL, BS * H)
        ).reshape(B, NL)
        eye_h = (
            lax.broadcasted_iota(jnp.int32, (H, NL), 0)
            == lax.broadcasted_iota(jnp.int32, (H, NL), 1) % H
        ).astype(jnp.float32)
        q2 = q_ref[...].reshape(B * H, D) * (D ** -0.5)
        k2 = k_ref[...].reshape(NL, D)
        v3 = v_ref[...].reshape(NL, D)

        s = lax.dot_general(
            q2, k2, (((1,), (1,)), ((), ())),
            preferred_element_type=jnp.float32,
        ).reshape(B, H, NL)
        m = jnp.max(s, axis=-1)
        p3 = (count3[:, None, :] * eye_h[None, :, :]
              * jnp.exp(s - m[:, :, None]))
        l = jnp.sum(p3, axis=-1)
        o = lax.dot_general(
            p3.reshape(B * H, NL), v3, (((1,), (0,)), ((), ())),
            preferred_element_type=jnp.float32,
        ).reshape(B, H, D)

        o_send[...] = o
        st_send[0, :, :] = m
        st_send[1, :, :] = l

        rdma_o = pltpu.make_async_remote_copy(
            src_ref=o_send, dst_ref=o_recv,
            send_sem=send_sems.at[0], recv_sem=recv_sems.at[0],
            device_id=nbr, device_id_type=pl.DeviceIdType.MESH,
        )
        rdma_st = pltpu.make_async_remote_copy(
            src_ref=st_send, dst_ref=st_recv,
            send_sem=send_sems.at[1], recv_sem=recv_sems.at[1],
            device_id=nbr, device_id_type=pl.DeviceIdType.MESH,
        )
        rdma_o.start()
        rdma_st.start()
        rdma_o.wait()
        rdma_st.wait()

        m1 = st_recv[0, :, :]
        l1 = st_recv[1, :, :]
        o1 = o_recv[...]
        m_new = jnp.maximum(m, m1)
        a0 = jnp.exp(m - m_new)
        a1 = jnp.exp(m1 - m_new)
        l_new = l * a0 + l1 * a1
        o_new = (o * a0[:, :, None] + o1 * a1[:, :, None]) / l_new[:, :, None]
        out_ref[...] = o_new.reshape(B, 1, H, D)

    return pl.pallas_call(
        body,
        out_shape=jax.ShapeDtypeStruct((B, 1, H, D), jnp.float32),
        in_specs=[pl.BlockSpec(memory_space=pltpu.VMEM)] * 5,
        out_specs=pl.BlockSpec(memory_space=pltpu.VMEM),
        scratch_shapes=[
            pltpu.VMEM((B, H, D), jnp.float32),
            pltpu.VMEM((B, H, D), jnp.float32),
            pltpu.VMEM((2, B, H), jnp.float32),
            pltpu.VMEM((2, B, H), jnp.float32),
            pltpu.SemaphoreType.DMA((2,)),
            pltpu.SemaphoreType.DMA((2,)),
        ],
        compiler_params=pltpu.CompilerParams(
            collective_id=0, vmem_limit_bytes=64 * 1024 * 1024
        ),
    )(Q, K, V, bt, lens.reshape(B, 1))
